# baseline (speedup 1.0000x reference)
import functools

import jax
import jax.numpy as jnp
from jax import lax
from jax.experimental import pallas as pl
from jax.experimental.pallas import tpu as pltpu
from jax.experimental.pallas import tpu_sc as plsc

DIM = 64
NUM_CORES = 2
NUM_SUBCORES = 16
LANES = 16
NUM_WORKERS = NUM_CORES * NUM_SUBCORES


def _make_kernel(batch):
    b_per_w = batch // NUM_WORKERS
    mesh = plsc.VectorSubcoreMesh(core_axis_name="c", subcore_axis_name="s")

    @functools.partial(
        pl.kernel,
        mesh=mesh,
        compiler_params=pltpu.CompilerParams(
            needs_layout_passes=False, use_tc_tiling_on_sc=False),
        out_type=jax.ShapeDtypeStruct((batch,), jnp.float32),
        scratch_types=[
            pltpu.VMEM((b_per_w,), jnp.int32),
            pltpu.VMEM((b_per_w,), jnp.int32),
            pltpu.VMEM((b_per_w, DIM), jnp.float32),
            pltpu.VMEM((b_per_w, DIM), jnp.float32),
            pltpu.VMEM((b_per_w,), jnp.float32),
            pltpu.SemaphoreType.DMA,
            pltpu.SemaphoreType.DMA,
        ],
    )
    def word2vec_sc(center_hbm, context_hbm, inw_hbm, outw_hbm, out_hbm,
                    cidx_v, xidx_v, v_rows, u_rows, res_v, sem_v, sem_u):
        wid = lax.axis_index("s") * NUM_CORES + lax.axis_index("c")
        base = wid * b_per_w
        pltpu.sync_copy(center_hbm.at[pl.ds(base, b_per_w)], cidx_v)
        pltpu.sync_copy(context_hbm.at[pl.ds(base, b_per_w)], xidx_v)
        cp_v = pltpu.async_copy(inw_hbm.at[cidx_v], v_rows, sem_v)
        cp_u = pltpu.async_copy(outw_hbm.at[xidx_v], u_rows, sem_u)
        cp_v.wait()
        cp_u.wait()

        n_chunks = DIM // LANES
        lane = lax.broadcasted_iota(jnp.int32, (LANES,), 0)
        lane_masks = [lane == j for j in range(LANES)]

        def group_body(g, _):
            base_row = g * LANES
            accv = jnp.zeros((LANES,), jnp.float32)
            for j in range(LANES):
                r = base_row + j
                acc = None
                for c in range(n_chunks):
                    vv = v_rows[r, pl.ds(c * LANES, LANES)]
                    uu = u_rows[r, pl.ds(c * LANES, LANES)]
                    p = vv * uu
                    acc = p if acc is None else acc + p
                accv = jnp.where(lane_masks[j], jnp.sum(acc), accv)
            res_v[pl.ds(base_row, LANES)] = accv
            return 0

        lax.fori_loop(0, b_per_w // LANES, group_body, 0)
        pltpu.sync_copy(res_v, out_hbm.at[pl.ds(base, b_per_w)])

    return word2vec_sc


def kernel(center_idx, context_idx, in_weight, out_weight):
    batch = center_idx.shape[0]
    fn = _make_kernel(batch)
    return fn(center_idx.astype(jnp.int32), context_idx.astype(jnp.int32),
              in_weight, out_weight)


# speedup vs baseline: 2.1805x; 2.1805x over previous
import functools

import jax
import jax.numpy as jnp
from jax import lax
from jax.experimental import pallas as pl
from jax.experimental.pallas import tpu as pltpu
from jax.experimental.pallas import tpu_sc as plsc

DIM = 64
TILE_ROWS = 8
NUM_CORES = 2
NUM_SUBCORES = 16
LANES = 16
NUM_WORKERS = NUM_CORES * NUM_SUBCORES
CHUNK = 32


def _make_kernel(batch):
    b_per_w = batch // NUM_WORKERS
    n_chunks = b_per_w // CHUNK
    mesh = plsc.VectorSubcoreMesh(core_axis_name="c", subcore_axis_name="s")

    @functools.partial(
        pl.kernel,
        mesh=mesh,
        compiler_params=pltpu.CompilerParams(needs_layout_passes=False),
        out_type=jax.ShapeDtypeStruct((batch,), jnp.float32),
        scratch_types=[
            pltpu.SMEM((b_per_w,), jnp.int32),
            pltpu.SMEM((b_per_w,), jnp.int32),
            pltpu.VMEM((b_per_w,), jnp.int32),
            pltpu.VMEM((CHUNK, TILE_ROWS, DIM), jnp.float32),
            pltpu.VMEM((CHUNK, TILE_ROWS, DIM), jnp.float32),
            pltpu.VMEM((b_per_w,), jnp.float32),
            pltpu.SemaphoreType.DMA,
            pltpu.SemaphoreType.DMA,
        ],
    )
    def word2vec_sc(center_hbm, context_hbm, inw_hbm, outw_hbm, out_hbm,
                    cidx_s, xidx_s, idx_v, v_slab, u_slab, res_v,
                    sem_v, sem_u):
        wid = lax.axis_index("s") * NUM_CORES + lax.axis_index("c")
        base = wid * b_per_w

        pltpu.sync_copy(center_hbm.at[pl.ds(base, b_per_w)], idx_v)

        def stage_c(g, _):
            vec = idx_v[pl.ds(g * LANES, LANES)]
            for j in range(LANES):
                cidx_s[g * LANES + j] = vec[j]
            return 0

        lax.fori_loop(0, b_per_w // LANES, stage_c, 0)
        pltpu.sync_copy(context_hbm.at[pl.ds(base, b_per_w)], idx_v)

        def stage_x(g, _):
            vec = idx_v[pl.ds(g * LANES, LANES)]
            for j in range(LANES):
                xidx_s[g * LANES + j] = vec[j]
            return 0

        lax.fori_loop(0, b_per_w // LANES, stage_x, 0)

        n_col = DIM // LANES
        lane = lax.broadcasted_iota(jnp.int32, (LANES,), 0)
        lane_masks = [lane == j for j in range(LANES)]

        def chunk_body(k, _):
            cbase = k * CHUNK
            for jj in range(CHUNK):
                ic = cidx_s[cbase + jj]
                ix = xidx_s[cbase + jj]
                pltpu.async_copy(inw_hbm.at[ic >> 3], v_slab.at[jj], sem_v)
                pltpu.async_copy(outw_hbm.at[ix >> 3], u_slab.at[jj], sem_u)
            pltpu.make_async_copy(inw_hbm.at[pl.ds(0, CHUNK)], v_slab,
                                  sem_v).wait()
            pltpu.make_async_copy(outw_hbm.at[pl.ds(0, CHUNK)], u_slab,
                                  sem_u).wait()
            for g in range(CHUNK // LANES):
                accv = jnp.zeros((LANES,), jnp.float32)
                for j in range(LANES):
                    b = g * LANES + j
                    rv = cidx_s[cbase + b] & (TILE_ROWS - 1)
                    ru = xidx_s[cbase + b] & (TILE_ROWS - 1)
                    acc = None
                    for c in range(n_col):
                        vv = v_slab[b, rv, pl.ds(c * LANES, LANES)]
                        uu = u_slab[b, ru, pl.ds(c * LANES, LANES)]
                        p = vv * uu
                        acc = p if acc is None else acc + p
                    accv = jnp.where(lane_masks[j], jnp.sum(acc), accv)
                res_v[pl.ds(cbase + g * LANES, LANES)] = accv
            return 0

        lax.fori_loop(0, n_chunks, chunk_body, 0)
        pltpu.sync_copy(res_v, out_hbm.at[pl.ds(base, b_per_w)])

    return word2vec_sc


def kernel(center_idx, context_idx, in_weight, out_weight):
    batch = center_idx.shape[0]
    vocab = in_weight.shape[0]
    fn = _make_kernel(batch)
    inw3 = in_weight.reshape(vocab // TILE_ROWS, TILE_ROWS, DIM)
    outw3 = out_weight.reshape(vocab // TILE_ROWS, TILE_ROWS, DIM)
    return fn(center_idx.astype(jnp.int32), context_idx.astype(jnp.int32),
              inw3, outw3)
